# baseline (speedup 1.0000x reference)
import functools

import jax
import jax.numpy as jnp
from jax.experimental import pallas as pl
from jax.experimental.pallas import tpu as pltpu

K_NEG = 25
K_NEIGH = K_NEG * 40
POS_MARGIN = 0.01
NEG_MARGIN = 2.0
NEG_PARAM = 0.2

N = 8192
D = 256


def _cx(key, idx, s, dirmask, lane):
    pk_m = pltpu.roll(key, -s, 1)
    pk_p = pltpu.roll(key, s, 1)
    pi_m = pltpu.roll(idx, -s, 1)
    pi_p = pltpu.roll(idx, s, 1)
    lower = (lane & s) == 0
    pk = jnp.where(lower, pk_m, pk_p)
    pi = jnp.where(lower, pi_m, pi_p)
    p_lt = (pk < key) | ((pk == key) & (pi < idx))
    p_eq = (pk == key) & (pi == idx)
    p_gt = (~p_lt) & (~p_eq)
    want_min = lower == dirmask
    take = (want_min & p_lt) | ((~want_min) & p_gt)
    return jnp.where(take, pk, key), jnp.where(take, pi, idx)


def _make_topk(n, d, c, br):
    log_c = c.bit_length() - 1
    assert 1 << log_c == c

    def body(a_ref, b_ref, an_ref, bn_ref, out_ref, keyb_ref, idxb_ref):
        i = pl.program_id(0)
        j = pl.program_id(1)
        nj = pl.num_programs(1)

        @pl.when(j == 0)
        def _init():
            keyb_ref[...] = jnp.full((br, c), jnp.inf, jnp.float32)
            idxb_ref[...] = jnp.zeros((br, c), jnp.int32)

        a = a_ref[...]
        b = b_ref[...]
        dot = jax.lax.dot_general(a, b, (((1,), (1,)), ((), ())))
        dist = an_ref[...].T + bn_ref[...] - 2.0 * dot
        dist = jnp.maximum(dist, 0.0)

        row_g = i * br + jax.lax.broadcasted_iota(jnp.int32, (br, c), 0)
        lane = jax.lax.broadcasted_iota(jnp.int32, (br, c), 1)
        col_g = j * c + lane
        key = jnp.where(row_g == col_g, jnp.inf, dist)
        idx = col_g

        def sort_outer(kk, carry):
            key, idx = carry
            k = jnp.int32(1) << kk

            def sort_inner(t, carry):
                key, idx = carry
                s = k >> (t + 1)
                dirmask = (lane & k) == 0
                return _cx(key, idx, s, dirmask, lane)

            return jax.lax.fori_loop(0, kk, sort_inner, (key, idx))

        key, idx = jax.lax.fori_loop(1, log_c + 1, sort_outer, (key, idx))

        bk = keyb_ref[...]
        bi = idxb_ref[...]
        t_lt = (key < bk) | ((key == bk) & (idx < bi))
        mk = jnp.where(t_lt, key, bk)
        mi = jnp.where(t_lt, idx, bi)

        asc = j == nj - 1
        dir_thr = jnp.where(asc, jnp.int32(c), jnp.int32(0))

        def merge_step(t, carry):
            key, idx = carry
            s = jnp.int32(c // 2) >> t
            dirmask = lane < dir_thr
            return _cx(key, idx, s, dirmask, lane)

        mk, mi = jax.lax.fori_loop(0, log_c, merge_step, (mk, mi))
        keyb_ref[...] = mk
        idxb_ref[...] = mi

        @pl.when(asc)
        def _out():
            out_ref[...] = mi

    def call(entity_emb, en, interpret=False):
        return pl.pallas_call(
            body,
            grid=(n // br, n // c),
            in_specs=[
                pl.BlockSpec((br, d), lambda i, j: (i, 0)),
                pl.BlockSpec((c, d), lambda i, j: (j, 0)),
                pl.BlockSpec((1, br), lambda i, j: (0, i)),
                pl.BlockSpec((1, c), lambda i, j: (0, j)),
            ],
            out_specs=pl.BlockSpec((br, c), lambda i, j: (i, 0)),
            out_shape=jax.ShapeDtypeStruct((n, c), jnp.int32),
            scratch_shapes=[
                pltpu.VMEM((br, c), jnp.float32),
                pltpu.VMEM((br, c), jnp.int32),
            ],
            interpret=interpret,
        )(entity_emb, entity_emb, en.T, en.T)

    return call


_topk_call = _make_topk(N, D, 1024, 256)


def kernel(entity_emb, rel_emb, pos_triples, neg_triples):
    en = jnp.sum(entity_emb * entity_emb, axis=1, keepdims=True)
    nn_idx = _topk_call(entity_emb, en)
    neighbours = nn_idx[:, :K_NEIGH]

    pos_head = jnp.take(entity_emb, pos_triples[:, 0], axis=0)
    pos_rel = jnp.take(rel_emb, pos_triples[:, 1], axis=0)
    pos_tail = jnp.take(entity_emb, pos_triples[:, 2], axis=0)
    neg_head = jnp.take(entity_emb, neg_triples[:, 0], axis=0)
    neg_rel = jnp.take(rel_emb, neg_triples[:, 1], axis=0)
    neg_tail = jnp.take(entity_emb, neg_triples[:, 2], axis=0)

    pos_score = jnp.sum((pos_head + pos_rel - pos_tail) ** 2, axis=1)
    neg_score = jnp.sum((neg_head + neg_rel - neg_tail) ** 2, axis=1)

    pos_loss = jax.nn.relu(pos_score - POS_MARGIN).sum()
    neg_loss = jax.nn.relu(NEG_MARGIN - neg_score).sum()
    loss = pos_loss + NEG_PARAM * neg_loss
    return loss, neighbours


# speedup vs baseline: 1.7465x; 1.7465x over previous
import functools

import jax
import jax.numpy as jnp
from jax.experimental import pallas as pl
from jax.experimental.pallas import tpu as pltpu

K_NEG = 25
K_NEIGH = K_NEG * 40
POS_MARGIN = 0.01
NEG_MARGIN = 2.0
NEG_PARAM = 0.2

N = 8192
D = 256


def _xor_partner(x, s, c, lane):
    if s >= 128:
        parts = []
        for b in range(c // (2 * s)):
            parts.append(x[:, (2 * b + 1) * s:(2 * b + 2) * s])
            parts.append(x[:, 2 * b * s:(2 * b + 1) * s])
        return jnp.concatenate(parts, axis=1)
    lower = (lane & s) == 0
    return jnp.where(lower, pltpu.roll(x, c - s, 1), pltpu.roll(x, s, 1))


def _cx(key, idx, s, dir_asc, c, lane):
    pk = _xor_partner(key, s, c, lane)
    pi = _xor_partner(idx, s, c, lane)
    lower = (lane & s) == 0
    want_min = ~(lower ^ dir_asc)
    eqk = pk == key
    p_lt = (pk < key) | (eqk & (pi < idx))
    p_le = p_lt | (eqk & (pi == idx))
    take = (want_min & p_lt) | ((~want_min) & (~p_le))
    return jnp.where(take, pk, key), jnp.where(take, pi, idx)


def _make_topk(n, d, c, br):
    log_c = c.bit_length() - 1
    assert 1 << log_c == c

    def body(a_ref, b_ref, an_ref, bn_ref, out_ref, keyb_ref, idxb_ref,
             dist_ref):
        i = pl.program_id(0)
        j = pl.program_id(1)
        nj = pl.num_programs(1)

        @pl.when(j == 0)
        def _init():
            keyb_ref[...] = jnp.full((br, c), jnp.inf, jnp.float32)
            idxb_ref[...] = jnp.zeros((br, c), jnp.int32)

        a = a_ref[...]
        b = b_ref[...]
        dot = jax.lax.dot_general(a, b, (((1,), (1,)), ((), ())))
        dist_ref[...] = jnp.maximum(an_ref[...] + bn_ref[...] - 2.0 * dot,
                                    0.0)

        sg = 16
        lane = jax.lax.broadcasted_iota(jnp.int32, (sg, c), 1)
        asc = lane >= 0
        for g in range(br // sg):
            r0 = g * sg
            dist = dist_ref[r0:r0 + sg, :]
            row_g = (i * br + r0
                     + jax.lax.broadcasted_iota(jnp.int32, (sg, c), 0))
            col_g = j * c + lane
            key = jnp.where(row_g == col_g, jnp.inf, dist)
            idx = col_g

            for kk in range(1, log_c + 1):
                k = 1 << kk
                dir_asc = (lane & k) != 0
                for t in range(kk):
                    s = k >> (t + 1)
                    key, idx = _cx(key, idx, s, dir_asc, c, lane)

            bk = keyb_ref[r0:r0 + sg, :]
            bi = idxb_ref[r0:r0 + sg, :]
            take_t = key < bk
            mk = jnp.where(take_t, key, bk)
            mi = jnp.where(take_t, idx, bi)

            for t in range(log_c):
                s = (c // 2) >> t
                mk, mi = _cx(mk, mi, s, asc, c, lane)

            keyb_ref[r0:r0 + sg, :] = mk
            idxb_ref[r0:r0 + sg, :] = mi

        @pl.when(j == nj - 1)
        def _out():
            out_ref[...] = idxb_ref[...]

    def call(entity_emb, en, interpret=False):
        return pl.pallas_call(
            body,
            grid=(n // br, n // c),
            in_specs=[
                pl.BlockSpec((br, d), lambda i, j: (i, 0)),
                pl.BlockSpec((c, d), lambda i, j: (j, 0)),
                pl.BlockSpec((br, 1), lambda i, j: (i, 0)),
                pl.BlockSpec((1, c), lambda i, j: (0, j)),
            ],
            out_specs=pl.BlockSpec((br, c), lambda i, j: (i, 0)),
            out_shape=jax.ShapeDtypeStruct((n, c), jnp.int32),
            scratch_shapes=[
                pltpu.VMEM((br, c), jnp.float32),
                pltpu.VMEM((br, c), jnp.int32),
                pltpu.VMEM((br, c), jnp.float32),
            ],
            interpret=interpret,
        )(entity_emb, entity_emb, en, en.T)

    return call


_topk_call = _make_topk(N, D, 1024, 32)


def kernel(entity_emb, rel_emb, pos_triples, neg_triples):
    en = jnp.sum(entity_emb * entity_emb, axis=1, keepdims=True)
    nn_idx = _topk_call(entity_emb, en)
    neighbours = nn_idx[:, :K_NEIGH]

    pos_head = jnp.take(entity_emb, pos_triples[:, 0], axis=0)
    pos_rel = jnp.take(rel_emb, pos_triples[:, 1], axis=0)
    pos_tail = jnp.take(entity_emb, pos_triples[:, 2], axis=0)
    neg_head = jnp.take(entity_emb, neg_triples[:, 0], axis=0)
    neg_rel = jnp.take(rel_emb, neg_triples[:, 1], axis=0)
    neg_tail = jnp.take(entity_emb, neg_triples[:, 2], axis=0)

    pos_score = jnp.sum((pos_head + pos_rel - pos_tail) ** 2, axis=1)
    neg_score = jnp.sum((neg_head + neg_rel - neg_tail) ** 2, axis=1)

    pos_loss = jax.nn.relu(pos_score - POS_MARGIN).sum()
    neg_loss = jax.nn.relu(NEG_MARGIN - neg_score).sum()
    loss = pos_loss + NEG_PARAM * neg_loss
    return loss, neighbours


# speedup vs baseline: 3.5181x; 2.0143x over previous
import functools

import jax
import jax.numpy as jnp
from jax.experimental import pallas as pl
from jax.experimental.pallas import tpu as pltpu

K_NEG = 25
K_NEIGH = K_NEG * 40
POS_MARGIN = 0.01
NEG_MARGIN = 2.0
NEG_PARAM = 0.2

N = 8192
D = 256


def _xor_partner(x, s, c, lane):
    if s >= 128:
        parts = []
        for b in range(c // (2 * s)):
            parts.append(x[:, (2 * b + 1) * s:(2 * b + 2) * s])
            parts.append(x[:, 2 * b * s:(2 * b + 1) * s])
        return jnp.concatenate(parts, axis=1)
    lower_i = ((lane & s) - 1) >> 31
    pp = pltpu.roll(x, s, 1)
    pm = pltpu.roll(x, c - s, 1)
    return pp ^ ((pp ^ pm) & lower_i)


def _cx(key, idx, s, dir_asc, c, lane):
    pk = _xor_partner(key, s, c, lane)
    pi = _xor_partner(idx, s, c, lane)
    lower = (lane & s) == 0
    want_min = ~(lower ^ dir_asc)
    eqk = pk == key
    p_lt = (pk < key) | (eqk & (pi < idx))
    take = ~(p_lt ^ want_min)
    return jnp.where(take, pk, key), jnp.where(take, pi, idx)


def _make_topk(n, d, c, br):
    log_c = c.bit_length() - 1
    assert 1 << log_c == c

    def body(a_ref, b_ref, an_ref, bn_ref, out_ref, keyb_ref, idxb_ref,
             dist_ref):
        i = pl.program_id(0)
        j = pl.program_id(1)
        nj = pl.num_programs(1)

        @pl.when(j == 0)
        def _init():
            keyb_ref[...] = jnp.full((br, c), 0x7F800000, jnp.int32)
            idxb_ref[...] = jnp.zeros((br, c), jnp.int32)

        a = a_ref[...]
        b = b_ref[...]
        dot = jax.lax.dot_general(a, b, (((1,), (1,)), ((), ())))
        dist_ref[...] = jnp.maximum(an_ref[...] + bn_ref[...] - 2.0 * dot,
                                    0.0)

        sg = 8
        ng = br // sg
        lane = jax.lax.broadcasted_iota(jnp.int32, (sg, c), 1)
        asc = lane >= 0
        keys, idxs = [], []
        for g in range(ng):
            r0 = g * sg
            dist = dist_ref[r0:r0 + sg, :]
            row_g = (i * br + r0
                     + jax.lax.broadcasted_iota(jnp.int32, (sg, c), 0))
            col_g = j * c + lane
            keys.append(jnp.where(row_g == col_g, jnp.int32(0x7F800000),
                                  pltpu.bitcast(dist, jnp.int32)))
            idxs.append(col_g)

        for kk in range(1, log_c + 1):
            k = 1 << kk
            dir_asc = (lane & k) != 0
            for t in range(kk):
                s = k >> (t + 1)
                for g in range(ng):
                    keys[g], idxs[g] = _cx(keys[g], idxs[g], s, dir_asc,
                                           c, lane)

        for g in range(ng):
            r0 = g * sg
            bk = keyb_ref[r0:r0 + sg, :]
            bi = idxb_ref[r0:r0 + sg, :]
            take_t = keys[g] < bk
            keys[g] = jnp.where(take_t, keys[g], bk)
            idxs[g] = jnp.where(take_t, idxs[g], bi)

        for t in range(log_c):
            s = (c // 2) >> t
            for g in range(ng):
                keys[g], idxs[g] = _cx(keys[g], idxs[g], s, asc, c, lane)

        for g in range(ng):
            r0 = g * sg
            keyb_ref[r0:r0 + sg, :] = keys[g]
            idxb_ref[r0:r0 + sg, :] = idxs[g]

        @pl.when(j == nj - 1)
        def _out():
            out_ref[...] = idxb_ref[...]

    def call(entity_emb, en, interpret=False):
        return pl.pallas_call(
            body,
            grid=(n // br, n // c),
            in_specs=[
                pl.BlockSpec((br, d), lambda i, j: (i, 0)),
                pl.BlockSpec((c, d), lambda i, j: (j, 0)),
                pl.BlockSpec((br, 1), lambda i, j: (i, 0)),
                pl.BlockSpec((1, c), lambda i, j: (0, j)),
            ],
            out_specs=pl.BlockSpec((br, c), lambda i, j: (i, 0)),
            out_shape=jax.ShapeDtypeStruct((n, c), jnp.int32),
            scratch_shapes=[
                pltpu.VMEM((br, c), jnp.int32),
                pltpu.VMEM((br, c), jnp.int32),
                pltpu.VMEM((br, c), jnp.float32),
            ],
            interpret=interpret,
        )(entity_emb, entity_emb, en, en.T)

    return call


_topk_call = _make_topk(N, D, 1024, 32)


def kernel(entity_emb, rel_emb, pos_triples, neg_triples):
    en = jnp.sum(entity_emb * entity_emb, axis=1, keepdims=True)
    nn_idx = _topk_call(entity_emb, en)
    neighbours = nn_idx[:, :K_NEIGH]

    pos_head = jnp.take(entity_emb, pos_triples[:, 0], axis=0)
    pos_rel = jnp.take(rel_emb, pos_triples[:, 1], axis=0)
    pos_tail = jnp.take(entity_emb, pos_triples[:, 2], axis=0)
    neg_head = jnp.take(entity_emb, neg_triples[:, 0], axis=0)
    neg_rel = jnp.take(rel_emb, neg_triples[:, 1], axis=0)
    neg_tail = jnp.take(entity_emb, neg_triples[:, 2], axis=0)

    pos_score = jnp.sum((pos_head + pos_rel - pos_tail) ** 2, axis=1)
    neg_score = jnp.sum((neg_head + neg_rel - neg_tail) ** 2, axis=1)

    pos_loss = jax.nn.relu(pos_score - POS_MARGIN).sum()
    neg_loss = jax.nn.relu(NEG_MARGIN - neg_score).sum()
    loss = pos_loss + NEG_PARAM * neg_loss
    return loss, neighbours


# speedup vs baseline: 3.8356x; 1.0903x over previous
import functools

import jax
import jax.numpy as jnp
from jax import lax
from jax.experimental import pallas as pl
from jax.experimental.pallas import tpu as pltpu
from jax.experimental.pallas import tpu_sc as plsc

K_NEG = 25
K_NEIGH = K_NEG * 40
POS_MARGIN = 0.01
NEG_MARGIN = 2.0
NEG_PARAM = 0.2

N = 8192
D = 256


def _xor_partner(x, s, c, lane):
    if s >= 128:
        parts = []
        for b in range(c // (2 * s)):
            parts.append(x[:, (2 * b + 1) * s:(2 * b + 2) * s])
            parts.append(x[:, 2 * b * s:(2 * b + 1) * s])
        return jnp.concatenate(parts, axis=1)
    lower_i = ((lane & s) - 1) >> 31
    pp = pltpu.roll(x, s, 1)
    pm = pltpu.roll(x, c - s, 1)
    return pp ^ ((pp ^ pm) & lower_i)


def _cx(key, idx, s, dir_asc, c, lane):
    pk = _xor_partner(key, s, c, lane)
    pi = _xor_partner(idx, s, c, lane)
    lower = (lane & s) == 0
    want_min = ~(lower ^ dir_asc)
    eqk = pk == key
    p_lt = (pk < key) | (eqk & (pi < idx))
    take = ~(p_lt ^ want_min)
    return jnp.where(take, pk, key), jnp.where(take, pi, idx)


def _make_topk(n, d, c, br):
    log_c = c.bit_length() - 1
    assert 1 << log_c == c

    def body(a_ref, b_ref, an_ref, bn_ref, out_ref, keyb_ref, idxb_ref,
             dist_ref):
        i = pl.program_id(0)
        j = pl.program_id(1)
        nj = pl.num_programs(1)

        @pl.when(j == 0)
        def _init():
            keyb_ref[...] = jnp.full((br, c), 0x7F800000, jnp.int32)
            idxb_ref[...] = jnp.zeros((br, c), jnp.int32)

        a = a_ref[...]
        b = b_ref[...]
        dot = jax.lax.dot_general(a, b, (((1,), (1,)), ((), ())))
        dist_ref[...] = jnp.maximum(an_ref[...] + bn_ref[...] - 2.0 * dot,
                                    0.0)

        sg = 8
        ng = br // sg
        lane = jax.lax.broadcasted_iota(jnp.int32, (sg, c), 1)
        asc = lane >= 0
        keys, idxs = [], []
        for g in range(ng):
            r0 = g * sg
            dist = dist_ref[r0:r0 + sg, :]
            row_g = (i * br + r0
                     + jax.lax.broadcasted_iota(jnp.int32, (sg, c), 0))
            col_g = j * c + lane
            keys.append(jnp.where(row_g == col_g, jnp.int32(0x7F800000),
                                  pltpu.bitcast(dist, jnp.int32)))
            idxs.append(col_g)

        for kk in range(1, log_c + 1):
            k = 1 << kk
            dir_asc = (lane & k) != 0
            for t in range(kk):
                s = k >> (t + 1)
                for g in range(ng):
                    keys[g], idxs[g] = _cx(keys[g], idxs[g], s, dir_asc,
                                           c, lane)

        for g in range(ng):
            r0 = g * sg
            bk = keyb_ref[r0:r0 + sg, :]
            bi = idxb_ref[r0:r0 + sg, :]
            take_t = keys[g] < bk
            keys[g] = jnp.where(take_t, keys[g], bk)
            idxs[g] = jnp.where(take_t, idxs[g], bi)

        for t in range(log_c):
            s = (c // 2) >> t
            for g in range(ng):
                keys[g], idxs[g] = _cx(keys[g], idxs[g], s, asc, c, lane)

        for g in range(ng):
            r0 = g * sg
            keyb_ref[r0:r0 + sg, :] = keys[g]
            idxb_ref[r0:r0 + sg, :] = idxs[g]

        @pl.when(j == nj - 1)
        def _out():
            out_ref[...] = idxb_ref[...]

    def call(entity_emb, en, interpret=False):
        return pl.pallas_call(
            body,
            grid=(n // br, n // c),
            in_specs=[
                pl.BlockSpec((br, d), lambda i, j: (i, 0)),
                pl.BlockSpec((c, d), lambda i, j: (j, 0)),
                pl.BlockSpec((br, 1), lambda i, j: (i, 0)),
                pl.BlockSpec((1, c), lambda i, j: (0, j)),
            ],
            out_specs=pl.BlockSpec((br, c), lambda i, j: (i, 0)),
            out_shape=jax.ShapeDtypeStruct((n, c), jnp.int32),
            scratch_shapes=[
                pltpu.VMEM((br, c), jnp.int32),
                pltpu.VMEM((br, c), jnp.int32),
                pltpu.VMEM((br, c), jnp.float32),
            ],
            interpret=interpret,
        )(entity_emb, entity_emb, en, en.T)

    return call


_topk_call = _make_topk(N, D, 1024, 64)

_SC_NC = 2
_SC_NS = 16
_SC_NW = _SC_NC * _SC_NS
_SC_L = 16


def _make_score(count, margin, is_pos):
    b = 64
    per_w = count // _SC_NW
    nch = per_w // b
    mesh = plsc.VectorSubcoreMesh(core_axis_name="c", subcore_axis_name="s")

    @functools.partial(
        pl.kernel, mesh=mesh,
        out_type=jax.ShapeDtypeStruct((_SC_NW, _SC_L), jnp.float32),
        scratch_types=[
            pltpu.VMEM((b,), jnp.int32),
            pltpu.VMEM((b,), jnp.int32),
            pltpu.VMEM((b,), jnp.int32),
            pltpu.VMEM((b, D), jnp.float32),
            pltpu.VMEM((b, D), jnp.float32),
            pltpu.VMEM((b, D), jnp.float32),
            pltpu.VMEM((_SC_L,), jnp.float32),
            pltpu.SemaphoreType.DMA,
        ],
    )
    def score(ent_hbm, rel_hbm, h_hbm, r_hbm, t_hbm, out_hbm,
              hidx, ridx, tidx, hrow, rrow, trow, ovec, sem):
        wid = lax.axis_index("s") * _SC_NC + lax.axis_index("c")
        base0 = wid * per_w

        def chunk(ci, acc):
            base = base0 + ci * b
            pltpu.sync_copy(h_hbm.at[pl.ds(base, b)], hidx)
            pltpu.sync_copy(r_hbm.at[pl.ds(base, b)], ridx)
            pltpu.sync_copy(t_hbm.at[pl.ds(base, b)], tidx)
            c1 = pltpu.async_copy(ent_hbm.at[hidx], hrow, sem)
            c2 = pltpu.async_copy(rel_hbm.at[ridx], rrow, sem)
            c3 = pltpu.async_copy(ent_hbm.at[tidx], trow, sem)
            c1.wait()
            c2.wait()
            c3.wait()

            def per_t(ti, acc2):
                def per_d(di, a16):
                    sl = pl.ds(di * _SC_L, _SC_L)
                    e = hrow[ti, sl] + rrow[ti, sl] - trow[ti, sl]
                    return a16 + e * e

                a16 = lax.fori_loop(
                    0, D // _SC_L, per_d,
                    jnp.zeros((_SC_L,), jnp.float32))
                s = a16[0]
                for q in range(1, _SC_L):
                    s = s + a16[q]
                if is_pos:
                    contrib = jnp.maximum(s - margin, 0.0)
                else:
                    contrib = jnp.maximum(margin - s, 0.0)
                return acc2 + contrib

            return lax.fori_loop(0, b, per_t, acc)

        acc = lax.fori_loop(0, nch, chunk, jnp.float32(0.0))
        ovec[...] = jnp.full((_SC_L,), acc, jnp.float32)
        pltpu.sync_copy(ovec, out_hbm.at[wid])

    return score


_pos_score = _make_score(P_COUNT := 4096, POS_MARGIN, True)
_neg_score = _make_score(Q_COUNT := 102400, NEG_MARGIN, False)


def kernel(entity_emb, rel_emb, pos_triples, neg_triples):
    en = jnp.sum(entity_emb * entity_emb, axis=1, keepdims=True)
    nn_idx = _topk_call(entity_emb, en)
    neighbours = nn_idx[:, :K_NEIGH]

    pos_part = _pos_score(entity_emb, rel_emb,
                          pos_triples[:, 0], pos_triples[:, 1],
                          pos_triples[:, 2])
    neg_part = _neg_score(entity_emb, rel_emb,
                          neg_triples[:, 0], neg_triples[:, 1],
                          neg_triples[:, 2])
    loss = pos_part[:, 0].sum() + NEG_PARAM * neg_part[:, 0].sum()
    return loss, neighbours


# speedup vs baseline: 4.7129x; 1.2287x over previous
import functools

import jax
import jax.numpy as jnp
from jax import lax
from jax.experimental import pallas as pl
from jax.experimental.pallas import tpu as pltpu
from jax.experimental.pallas import tpu_sc as plsc

K_NEG = 25
K_NEIGH = K_NEG * 40
POS_MARGIN = 0.01
NEG_MARGIN = 2.0
NEG_PARAM = 0.2

N = 8192
D = 256


def _xor_partner(x, s, c, lane):
    if s >= 128:
        parts = []
        for b in range(c // (2 * s)):
            parts.append(x[:, (2 * b + 1) * s:(2 * b + 2) * s])
            parts.append(x[:, 2 * b * s:(2 * b + 1) * s])
        return jnp.concatenate(parts, axis=1)
    lower_i = ((lane & s) - 1) >> 31
    pp = pltpu.roll(x, s, 1)
    pm = pltpu.roll(x, c - s, 1)
    return pp ^ ((pp ^ pm) & lower_i)


def _cx(key, idx, s, dir_asc, c, lane):
    pk = _xor_partner(key, s, c, lane)
    pi = _xor_partner(idx, s, c, lane)
    lower = (lane & s) == 0
    want_min = ~(lower ^ dir_asc)
    eqk = pk == key
    p_lt = (pk < key) | (eqk & (pi < idx))
    take = ~(p_lt ^ want_min)
    return jnp.where(take, pk, key), jnp.where(take, pi, idx)


def _make_topk(n, d, c, br):
    log_c = c.bit_length() - 1
    assert 1 << log_c == c

    def body(a_ref, b_ref, an_ref, bn_ref, out_ref, keyb_ref, idxb_ref,
             dist_ref):
        i = pl.program_id(0)
        j = pl.program_id(1)
        nj = pl.num_programs(1)

        @pl.when(j == 0)
        def _init():
            keyb_ref[...] = jnp.full((br, c), 0x7F800000, jnp.int32)
            idxb_ref[...] = jnp.zeros((br, c), jnp.int32)

        a = a_ref[...]
        b = b_ref[...]
        dot = jax.lax.dot_general(a, b, (((1,), (1,)), ((), ())))
        dist_ref[...] = jnp.maximum(an_ref[...] + bn_ref[...] - 2.0 * dot,
                                    0.0)

        sg = 8
        ng = br // sg
        lane = jax.lax.broadcasted_iota(jnp.int32, (sg, c), 1)
        asc = lane >= 0
        keys, idxs = [], []
        for g in range(ng):
            r0 = g * sg
            dist = dist_ref[r0:r0 + sg, :]
            row_g = (i * br + r0
                     + jax.lax.broadcasted_iota(jnp.int32, (sg, c), 0))
            col_g = j * c + lane
            keys.append(jnp.where(row_g == col_g, jnp.int32(0x7F800000),
                                  pltpu.bitcast(dist, jnp.int32)))
            idxs.append(col_g)

        for kk in range(1, log_c + 1):
            kbit = (1 << (log_c - 1 - kk)) if kk < log_c else 0
            dir_asc = (lane & kbit) != 0
            for t in range(kk):
                s = 1 << (log_c - kk + t)
                for g in range(ng):
                    keys[g], idxs[g] = _cx(keys[g], idxs[g], s, dir_asc,
                                           c, lane)

        for g in range(ng):
            r0 = g * sg
            bk = keyb_ref[r0:r0 + sg, :]
            bi = idxb_ref[r0:r0 + sg, :]
            take_t = keys[g] < bk
            keys[g] = jnp.where(take_t, keys[g], bk)
            idxs[g] = jnp.where(take_t, idxs[g], bi)

        for t in range(log_c):
            s = 1 << t
            for g in range(ng):
                keys[g], idxs[g] = _cx(keys[g], idxs[g], s, asc, c, lane)

        for g in range(ng):
            r0 = g * sg
            keyb_ref[r0:r0 + sg, :] = keys[g]
            idxb_ref[r0:r0 + sg, :] = idxs[g]

        @pl.when(j == nj - 1)
        def _out():
            for g in range(ng):
                x = idxs[g]
                for abit in range(log_c // 2):
                    bbit = log_c - 1 - abit
                    if abit >= bbit:
                        break
                    d1 = (1 << bbit) - (1 << abit)
                    hi_only = ((lane & (1 << bbit)) != 0) & \
                        ((lane & (1 << abit)) == 0)
                    lo_only = ((lane & (1 << abit)) != 0) & \
                        ((lane & (1 << bbit)) == 0)
                    x = jnp.where(hi_only, pltpu.roll(x, d1, 1),
                                  jnp.where(lo_only, pltpu.roll(x, c - d1, 1),
                                            x))
                r0 = g * sg
                out_ref[r0:r0 + sg, :] = x

    def call(entity_emb, en, interpret=False):
        return pl.pallas_call(
            body,
            grid=(n // br, n // c),
            in_specs=[
                pl.BlockSpec((br, d), lambda i, j: (i, 0)),
                pl.BlockSpec((c, d), lambda i, j: (j, 0)),
                pl.BlockSpec((br, 1), lambda i, j: (i, 0)),
                pl.BlockSpec((1, c), lambda i, j: (0, j)),
            ],
            out_specs=pl.BlockSpec((br, c), lambda i, j: (i, 0)),
            out_shape=jax.ShapeDtypeStruct((n, c), jnp.int32),
            scratch_shapes=[
                pltpu.VMEM((br, c), jnp.int32),
                pltpu.VMEM((br, c), jnp.int32),
                pltpu.VMEM((br, c), jnp.float32),
            ],
            interpret=interpret,
        )(entity_emb, entity_emb, en, en.T)

    return call


_topk_call = _make_topk(N, D, 1024, 64)

_SC_NC = 2
_SC_NS = 16
_SC_NW = _SC_NC * _SC_NS
_SC_L = 16


def _make_score(count, margin, is_pos):
    b = 64
    per_w = count // _SC_NW
    nch = per_w // b
    mesh = plsc.VectorSubcoreMesh(core_axis_name="c", subcore_axis_name="s")

    @functools.partial(
        pl.kernel, mesh=mesh,
        out_type=jax.ShapeDtypeStruct((_SC_NW, _SC_L), jnp.float32),
        scratch_types=[
            pltpu.VMEM((b,), jnp.int32),
            pltpu.VMEM((b,), jnp.int32),
            pltpu.VMEM((b,), jnp.int32),
            pltpu.VMEM((b, D), jnp.float32),
            pltpu.VMEM((b, D), jnp.float32),
            pltpu.VMEM((b, D), jnp.float32),
            pltpu.VMEM((_SC_L,), jnp.float32),
            pltpu.SemaphoreType.DMA,
        ],
    )
    def score(ent_hbm, rel_hbm, h_hbm, r_hbm, t_hbm, out_hbm,
              hidx, ridx, tidx, hrow, rrow, trow, ovec, sem):
        wid = lax.axis_index("s") * _SC_NC + lax.axis_index("c")
        base0 = wid * per_w

        def chunk(ci, acc):
            base = base0 + ci * b
            pltpu.sync_copy(h_hbm.at[pl.ds(base, b)], hidx)
            pltpu.sync_copy(r_hbm.at[pl.ds(base, b)], ridx)
            pltpu.sync_copy(t_hbm.at[pl.ds(base, b)], tidx)
            c1 = pltpu.async_copy(ent_hbm.at[hidx], hrow, sem)
            c2 = pltpu.async_copy(rel_hbm.at[ridx], rrow, sem)
            c3 = pltpu.async_copy(ent_hbm.at[tidx], trow, sem)
            c1.wait()
            c2.wait()
            c3.wait()

            def per_t(ti, acc2):
                def per_d(di, a16):
                    sl = pl.ds(di * _SC_L, _SC_L)
                    e = hrow[ti, sl] + rrow[ti, sl] - trow[ti, sl]
                    return a16 + e * e

                a16 = lax.fori_loop(
                    0, D // _SC_L, per_d,
                    jnp.zeros((_SC_L,), jnp.float32))
                s = a16[0]
                for q in range(1, _SC_L):
                    s = s + a16[q]
                if is_pos:
                    contrib = jnp.maximum(s - margin, 0.0)
                else:
                    contrib = jnp.maximum(margin - s, 0.0)
                return acc2 + contrib

            return lax.fori_loop(0, b, per_t, acc)

        acc = lax.fori_loop(0, nch, chunk, jnp.float32(0.0))
        ovec[...] = jnp.full((_SC_L,), acc, jnp.float32)
        pltpu.sync_copy(ovec, out_hbm.at[wid])

    return score


@functools.lru_cache(maxsize=None)
def _get_scorers():
    return (_make_score(4096, POS_MARGIN, True),
            _make_score(102400, NEG_MARGIN, False))


def kernel(entity_emb, rel_emb, pos_triples, neg_triples):
    en = jnp.sum(entity_emb * entity_emb, axis=1, keepdims=True)
    nn_idx = _topk_call(entity_emb, en)
    neighbours = nn_idx[:, :K_NEIGH]

    pos_score_k, neg_score_k = _get_scorers()
    pos_part = pos_score_k(entity_emb, rel_emb,
                           pos_triples[:, 0], pos_triples[:, 1],
                           pos_triples[:, 2])
    neg_part = neg_score_k(entity_emb, rel_emb,
                           neg_triples[:, 0], neg_triples[:, 1],
                           neg_triples[:, 2])
    loss = pos_part[:, 0].sum() + NEG_PARAM * neg_part[:, 0].sum()
    return loss, neighbours
